# speedup vs baseline: 2.5360x; 2.5360x over previous
import functools

import jax
import jax.numpy as jnp
from jax import lax
from jax.experimental import pallas as pl
from jax.experimental.pallas import tpu as pltpu
from jax.experimental.pallas import tpu_sc as plsc

NUM_OPS = 1000
NUM_BW = 32
EMB = 128
BATCH = 16384


def _tables_body(opt_ref, pt_ref, w1_ref, b1_ref, top_ref, t0_ref, t1_ref,
                 t2_ref):
    w1 = w1_ref[...]
    top_ref[...] = jnp.dot(opt_ref[...], w1[0:128],
                           preferred_element_type=jnp.float32) + b1_ref[...]
    pt = pt_ref[...]
    t0_ref[...] = jnp.dot(pt, w1[128:256], preferred_element_type=jnp.float32)
    t1_ref[...] = jnp.dot(pt, w1[256:384], preferred_element_type=jnp.float32)
    t2_ref[...] = jnp.dot(pt, w1[384:512], preferred_element_type=jnp.float32)


def _transform_tables(op_table, param_table, W1, b1):
    return pl.pallas_call(
        _tables_body,
        out_shape=(
            jax.ShapeDtypeStruct((NUM_OPS, EMB), jnp.float32),
            jax.ShapeDtypeStruct((NUM_BW, EMB), jnp.float32),
            jax.ShapeDtypeStruct((NUM_BW, EMB), jnp.float32),
            jax.ShapeDtypeStruct((NUM_BW, EMB), jnp.float32),
        ),
    )(op_table, param_table, W1, b1.reshape(1, EMB))


def _sc_gather_sum(op, p0, p1, p2, top, t0, t1, t2):
    info = plsc.get_sparse_core_info()
    nc, ns = info.num_cores, info.num_subcores
    nw = nc * ns
    bpw = BATCH // nw
    ch = 128
    steps = bpw // ch

    mesh = plsc.VectorSubcoreMesh(core_axis_name="c", subcore_axis_name="s")

    @functools.partial(
        pl.kernel,
        mesh=mesh,
        out_type=jax.ShapeDtypeStruct((BATCH, EMB), jnp.float32),
        scratch_types=[
            pltpu.VMEM((ch,), jnp.int32),
            pltpu.VMEM((ch,), jnp.int32),
            pltpu.VMEM((ch,), jnp.int32),
            pltpu.VMEM((ch,), jnp.int32),
            pltpu.VMEM((ch, EMB), jnp.float32),
            pltpu.VMEM((ch, EMB), jnp.float32),
            pltpu.VMEM((ch, EMB), jnp.float32),
            pltpu.VMEM((ch, EMB), jnp.float32),
            pltpu.VMEM((ch, EMB), jnp.float32),
            pltpu.SemaphoreType.DMA,
        ],
    )
    def k(op_hbm, p0_hbm, p1_hbm, p2_hbm, top_hbm, t0_hbm, t1_hbm, t2_hbm,
          out_hbm, iop, i0, i1, i2, r0, r1, r2, r3, ob, sem):
        wid = lax.axis_index("s") * nc + lax.axis_index("c")
        base0 = wid * bpw
        for s in range(steps):
            base = base0 + s * ch
            pltpu.sync_copy(op_hbm.at[pl.ds(base, ch)], iop)
            pltpu.sync_copy(p0_hbm.at[pl.ds(base, ch)], i0)
            pltpu.sync_copy(p1_hbm.at[pl.ds(base, ch)], i1)
            pltpu.sync_copy(p2_hbm.at[pl.ds(base, ch)], i2)
            c0 = pltpu.async_copy(top_hbm.at[iop], r0, sem)
            c1 = pltpu.async_copy(t0_hbm.at[i0], r1, sem)
            c2 = pltpu.async_copy(t1_hbm.at[i1], r2, sem)
            c3 = pltpu.async_copy(t2_hbm.at[i2], r3, sem)
            c0.wait()
            c1.wait()
            c2.wait()
            c3.wait()

            def row(r, _):
                for c in range(EMB // 16):
                    sl = pl.ds(c * 16, 16)
                    ob[r, sl] = r0[r, sl] + r1[r, sl] + r2[r, sl] + r3[r, sl]
                return 0

            lax.fori_loop(0, ch, row, 0)
            pltpu.sync_copy(ob, out_hbm.at[pl.ds(base, ch)])

    return k(op, p0, p1, p2, top, t0, t1, t2)


def _mlp2_body(h_ref, w2_ref, b2_ref, o_ref):
    h = jnp.maximum(h_ref[...], 0.0)
    o_ref[...] = jnp.dot(h, w2_ref[...],
                         preferred_element_type=jnp.float32) + b2_ref[...]


def _mlp2(h_pre, W2, b2):
    blk = 1024
    return pl.pallas_call(
        _mlp2_body,
        grid=(BATCH // blk,),
        in_specs=[
            pl.BlockSpec((blk, EMB), lambda i: (i, 0)),
            pl.BlockSpec((EMB, EMB), lambda i: (0, 0)),
            pl.BlockSpec((1, EMB), lambda i: (0, 0)),
        ],
        out_specs=pl.BlockSpec((blk, EMB), lambda i: (i, 0)),
        out_shape=jax.ShapeDtypeStruct((BATCH, EMB), jnp.float32),
    )(h_pre, W2, b2.reshape(1, EMB))


def kernel(op, params, op_table, param_table, W1, b1, W2, b2):
    op = op.astype(jnp.int32)
    params = params.astype(jnp.int32)
    top, t0, t1, t2 = _transform_tables(op_table, param_table, W1, b1)
    h_pre = _sc_gather_sum(op, params[0], params[1], params[2],
                           top, t0, t1, t2)
    return _mlp2(h_pre, W2, b2)


# speedup vs baseline: 4.9752x; 1.9618x over previous
import functools

import jax
import jax.numpy as jnp
from jax import lax
from jax.experimental import pallas as pl
from jax.experimental.pallas import tpu as pltpu
from jax.experimental.pallas import tpu_sc as plsc

NUM_OPS = 1000
NUM_BW = 32
EMB = 128
BATCH = 16384


def _tables_body(opt_ref, pt_ref, w1_ref, b1_ref, top_ref, t0_ref, t1_ref,
                 t2_ref):
    w1 = w1_ref[...]
    top_ref[...] = jnp.dot(opt_ref[...], w1[0:128],
                           preferred_element_type=jnp.float32) + b1_ref[...]
    pt = pt_ref[...]
    t0_ref[...] = jnp.dot(pt, w1[128:256], preferred_element_type=jnp.float32)
    t1_ref[...] = jnp.dot(pt, w1[256:384], preferred_element_type=jnp.float32)
    t2_ref[...] = jnp.dot(pt, w1[384:512], preferred_element_type=jnp.float32)


def _transform_tables(op_table, param_table, W1, b1):
    return pl.pallas_call(
        _tables_body,
        out_shape=(
            jax.ShapeDtypeStruct((NUM_OPS, EMB), jnp.float32),
            jax.ShapeDtypeStruct((NUM_BW, EMB), jnp.float32),
            jax.ShapeDtypeStruct((NUM_BW, EMB), jnp.float32),
            jax.ShapeDtypeStruct((NUM_BW, EMB), jnp.float32),
        ),
    )(op_table, param_table, W1, b1.reshape(1, EMB))


def _sc_gather(op2d, top):
    info = plsc.get_sparse_core_info()
    nc, ns = info.num_cores, info.num_subcores
    nw = nc * ns
    ch = 128
    nchunks = BATCH // ch
    steps = nchunks // nw

    mesh = plsc.VectorSubcoreMesh(core_axis_name="c", subcore_axis_name="s")

    @functools.partial(
        pl.kernel,
        mesh=mesh,
        out_type=jax.ShapeDtypeStruct((BATCH, EMB), jnp.float32),
        scratch_types=[
            pltpu.VMEM((steps, ch), jnp.int32),
            pltpu.VMEM((ch, EMB), jnp.float32),
            pltpu.VMEM((ch, EMB), jnp.float32),
            pltpu.SemaphoreType.DMA,
            pltpu.SemaphoreType.DMA,
        ],
    )
    def k(op_hbm, top_hbm, out_hbm, idx, b0, b1_, gsem, ssem):
        wid = lax.axis_index("s") * nc + lax.axis_index("c")
        chunk0 = wid * steps
        for s in range(steps):
            pltpu.sync_copy(op_hbm.at[chunk0 + s], idx.at[s])
        bufs = (b0, b1_)
        gathers = [None] * steps
        stores = [None] * steps
        gathers[0] = pltpu.async_copy(top_hbm.at[idx.at[0]], bufs[0], gsem)
        for s in range(steps):
            if s + 1 < steps:
                if stores[s - 1] is not None:
                    stores[s - 1].wait()
                gathers[s + 1] = pltpu.async_copy(
                    top_hbm.at[idx.at[s + 1]], bufs[(s + 1) % 2], gsem)
            gathers[s].wait()
            base = (chunk0 + s) * ch
            stores[s] = pltpu.async_copy(
                bufs[s % 2], out_hbm.at[pl.ds(base, ch)], ssem)
        stores[steps - 1].wait()
        stores[steps - 2].wait()

    return k(op2d, top)


def _mlp2_body(h_ref, p_ref, t0_ref, t1_ref, t2_ref, w2_ref, b2_ref, o_ref):
    blk = h_ref.shape[0]
    h = h_ref[...]
    for i, t_ref in enumerate((t0_ref, t1_ref, t2_ref)):
        pi = p_ref[0, i, :]
        oh = (pi[:, None] == lax.broadcasted_iota(jnp.int32, (blk, NUM_BW), 1)
              ).astype(jnp.float32)
        h = h + jnp.dot(oh, t_ref[...], preferred_element_type=jnp.float32)
    h = jnp.maximum(h, 0.0)
    o_ref[...] = jnp.dot(h, w2_ref[...],
                         preferred_element_type=jnp.float32) + b2_ref[...]


def _mlp2(h_op, params3d, t0, t1, t2, W2, b2):
    blk = 1024
    nblk = BATCH // blk
    return pl.pallas_call(
        _mlp2_body,
        grid=(nblk,),
        in_specs=[
            pl.BlockSpec((blk, EMB), lambda i: (i, 0)),
            pl.BlockSpec((1, 3, blk), lambda i: (i, 0, 0)),
            pl.BlockSpec((NUM_BW, EMB), lambda i: (0, 0)),
            pl.BlockSpec((NUM_BW, EMB), lambda i: (0, 0)),
            pl.BlockSpec((NUM_BW, EMB), lambda i: (0, 0)),
            pl.BlockSpec((EMB, EMB), lambda i: (0, 0)),
            pl.BlockSpec((1, EMB), lambda i: (0, 0)),
        ],
        out_specs=pl.BlockSpec((blk, EMB), lambda i: (i, 0)),
        out_shape=jax.ShapeDtypeStruct((BATCH, EMB), jnp.float32),
    )(h_op, params3d, t0, t1, t2, W2, b2.reshape(1, EMB))


def kernel(op, params, op_table, param_table, W1, b1, W2, b2):
    op2d = op.astype(jnp.int32).reshape(BATCH // 128, 128)
    params3d = params.astype(jnp.int32).reshape(3, BATCH // 1024, 1024)
    params3d = params3d.transpose(1, 0, 2)
    top, t0, t1, t2 = _transform_tables(op_table, param_table, W1, b1)
    h_op = _sc_gather(op2d, top)
    return _mlp2(h_op, params3d, t0, t1, t2, W2, b2)


# speedup vs baseline: 5.7540x; 1.1565x over previous
import functools

import jax
import jax.numpy as jnp
from jax import lax
from jax.experimental import pallas as pl
from jax.experimental.pallas import tpu as pltpu
from jax.experimental.pallas import tpu_sc as plsc

NUM_OPS = 1000
NUM_BW = 32
EMB = 128
BATCH = 16384


def _tables_body(opt_ref, pt_ref, w1_ref, b1_ref, top_ref, tcat_ref):
    w1 = w1_ref[...]
    top_ref[...] = jnp.dot(opt_ref[...], w1[0:128],
                           preferred_element_type=jnp.float32) + b1_ref[...]
    pt = pt_ref[...]
    tcat_ref[0:32] = jnp.dot(pt, w1[128:256],
                             preferred_element_type=jnp.float32)
    tcat_ref[32:64] = jnp.dot(pt, w1[256:384],
                              preferred_element_type=jnp.float32)
    tcat_ref[64:96] = jnp.dot(pt, w1[384:512],
                              preferred_element_type=jnp.float32)


def _transform_tables(op_table, param_table, W1, b1):
    return pl.pallas_call(
        _tables_body,
        out_shape=(
            jax.ShapeDtypeStruct((NUM_OPS, EMB), jnp.float32),
            jax.ShapeDtypeStruct((3 * NUM_BW, EMB), jnp.float32),
        ),
    )(op_table, param_table, W1, b1.reshape(1, EMB))


def _sc_gather(op2d, top):
    info = plsc.get_sparse_core_info()
    nc, ns = info.num_cores, info.num_subcores
    nw = nc * ns
    ch = 128
    nchunks = BATCH // ch
    steps = nchunks // nw

    mesh = plsc.VectorSubcoreMesh(core_axis_name="c", subcore_axis_name="s")

    @functools.partial(
        pl.kernel,
        mesh=mesh,
        out_type=jax.ShapeDtypeStruct((BATCH, EMB), jnp.float32),
        scratch_types=[
            pltpu.VMEM((steps, ch), jnp.int32),
            pltpu.VMEM((ch, EMB), jnp.float32),
            pltpu.VMEM((ch, EMB), jnp.float32),
            pltpu.SemaphoreType.DMA,
            pltpu.SemaphoreType.DMA,
        ],
    )
    def k(op_hbm, top_hbm, out_hbm, idx, b0, b1_, gsem, ssem):
        wid = lax.axis_index("s") * nc + lax.axis_index("c")
        chunk0 = wid * steps
        for s in range(steps):
            pltpu.sync_copy(op_hbm.at[chunk0 + s], idx.at[s])
        bufs = (b0, b1_)
        gathers = [None] * steps
        stores = [None] * steps
        gathers[0] = pltpu.async_copy(top_hbm.at[idx.at[0]], bufs[0], gsem)
        for s in range(steps):
            if s + 1 < steps:
                if stores[s - 1] is not None:
                    stores[s - 1].wait()
                gathers[s + 1] = pltpu.async_copy(
                    top_hbm.at[idx.at[s + 1]], bufs[(s + 1) % 2], gsem)
            gathers[s].wait()
            base = (chunk0 + s) * ch
            stores[s] = pltpu.async_copy(
                bufs[s % 2], out_hbm.at[pl.ds(base, ch)], ssem)
        stores[steps - 1].wait()
        stores[steps - 2].wait()

    return k(op2d, top)


def _mlp2_body(h_ref, p_ref, tcat_ref, w2_ref, b2_ref, o_ref):
    blk = h_ref.shape[0]
    p = p_ref[0]
    io = lax.broadcasted_iota(jnp.int32, (3 * NUM_BW, blk), 0)
    hit = ((io == p[0:1, :]) | (io == p[1:2, :]) | (io == p[2:3, :]))
    oh_t = hit.astype(jnp.float32)
    hp = lax.dot_general(oh_t, tcat_ref[...],
                         (((0,), (0,)), ((), ())),
                         preferred_element_type=jnp.float32)
    h = jnp.maximum(h_ref[...] + hp, 0.0)
    o_ref[...] = jnp.dot(h, w2_ref[...],
                         preferred_element_type=jnp.float32) + b2_ref[...]


def _mlp2(h_op, params3d, tcat, W2, b2):
    blk = 2048
    nblk = BATCH // blk
    return pl.pallas_call(
        _mlp2_body,
        grid=(nblk,),
        in_specs=[
            pl.BlockSpec((blk, EMB), lambda i: (i, 0)),
            pl.BlockSpec((1, 3, blk), lambda i: (i, 0, 0)),
            pl.BlockSpec((3 * NUM_BW, EMB), lambda i: (0, 0)),
            pl.BlockSpec((EMB, EMB), lambda i: (0, 0)),
            pl.BlockSpec((1, EMB), lambda i: (0, 0)),
        ],
        out_specs=pl.BlockSpec((blk, EMB), lambda i: (i, 0)),
        out_shape=jax.ShapeDtypeStruct((BATCH, EMB), jnp.float32),
    )(h_op, params3d, tcat, W2, b2.reshape(1, EMB))


def kernel(op, params, op_table, param_table, W1, b1, W2, b2):
    blk = 2048
    op2d = op.astype(jnp.int32).reshape(BATCH // 128, 128)
    shift = jnp.array([0, NUM_BW, 2 * NUM_BW], dtype=jnp.int32)[:, None]
    params3d = (params.astype(jnp.int32) + shift).reshape(
        3, BATCH // blk, blk).transpose(1, 0, 2)
    top, tcat = _transform_tables(op_table, param_table, W1, b1)
    h_op = _sc_gather(op2d, top)
    return _mlp2(h_op, params3d, tcat, W2, b2)
